# Initial kernel scaffold; baseline (speedup 1.0000x reference)
import functools

import jax
import jax.numpy as jnp
from jax import lax
from jax.experimental import pallas as pl
from jax.experimental.pallas import tpu as pltpu
from jax.experimental.pallas import tpu_sc as plsc

N = 10000
D = 128
E = 320000
REG_PARAM = 0.01

NC, NS = 2, 16
NW = NC * NS
EPW = E // NW
CH = 80
NCHUNK = EPW // CH


def _sc_scores_body(u_hbm, i_hbm, srcc_hbm, dstc_hbm, srcb_hbm, dstb_hbm,
                    wc_hbm, wb_hbm, outc_hbm, outb_hbm,
                    idxs, idxd, urows, irows, wbuf, scores, sem):
    wid = lax.axis_index("s") * NC + lax.axis_index("c")
    base = wid * EPW

    def run_etype(src_hbm, dst_hbm, w_hbm, out_hbm):
        pltpu.sync_copy(w_hbm, wbuf)
        wv = [wbuf[pl.ds(16 * j, 16)] for j in range(8)]

        def chunk_body(c, carry):
            off = base + c * CH
            pltpu.sync_copy(src_hbm.at[pl.ds(off, CH)], idxs)
            pltpu.sync_copy(dst_hbm.at[pl.ds(off, CH)], idxd)
            cu = pltpu.async_copy(u_hbm.at[idxs], urows, sem)
            ci = pltpu.async_copy(i_hbm.at[idxd], irows, sem)
            cu.wait()
            ci.wait()

            def edge_body(e, carry2):
                acc = urows[e, pl.ds(0, 16)] * wv[0] * irows[e, pl.ds(0, 16)]
                for j in range(1, 8):
                    acc = acc + (urows[e, pl.ds(16 * j, 16)] * wv[j]
                                 * irows[e, pl.ds(16 * j, 16)])
                scores[c * CH + e] = jnp.sum(acc)
                return carry2

            lax.fori_loop(0, CH, edge_body, 0)
            return carry

        lax.fori_loop(0, NCHUNK, chunk_body, 0)
        pltpu.sync_copy(scores, out_hbm.at[pl.ds(base, EPW)])

    run_etype(srcc_hbm, dstc_hbm, wc_hbm, outc_hbm)
    run_etype(srcb_hbm, dstb_hbm, wb_hbm, outb_hbm)


_sc_scores = pl.kernel(
    _sc_scores_body,
    out_type=(jax.ShapeDtypeStruct((E,), jnp.float32),
              jax.ShapeDtypeStruct((E,), jnp.float32)),
    mesh=plsc.VectorSubcoreMesh(core_axis_name="c", subcore_axis_name="s",
                                num_cores=NC, num_subcores=NS),
    scratch_types=[
        pltpu.VMEM((CH,), jnp.int32),
        pltpu.VMEM((CH,), jnp.int32),
        pltpu.VMEM((CH, D), jnp.float32),
        pltpu.VMEM((CH, D), jnp.float32),
        pltpu.VMEM((D,), jnp.float32),
        pltpu.VMEM((EPW,), jnp.float32),
        pltpu.SemaphoreType.DMA,
    ],
)


def _tc_loss_body(sc_ref, sb_ref, lc_ref, lb_ref, u_ref, i_ref, wc_ref, wb_ref,
                  out_ref):
    def bce_sum(s, y):
        return jnp.sum(jnp.maximum(s, 0.0) - s * y
                       + jnp.log1p(jnp.exp(-jnp.abs(s))))

    predict = (bce_sum(sc_ref[...], lc_ref[...])
               + bce_sum(sb_ref[...], lb_ref[...])) / E
    reg = (jnp.mean(u_ref[...] ** 2) + jnp.mean(i_ref[...] ** 2)
           + jnp.mean(wc_ref[...] ** 2) + jnp.mean(wb_ref[...] ** 2))
    out_ref[...] = jnp.full((1, 1), predict + REG_PARAM * reg, jnp.float32)


_tc_loss = pl.pallas_call(
    _tc_loss_body,
    out_shape=jax.ShapeDtypeStruct((1, 1), jnp.float32),
)


def kernel(embed_user, embed_item, edges_click, edges_buy, labels_click,
           labels_buy, w_click, w_buy):
    srcc = edges_click[:, 0]
    dstc = edges_click[:, 1]
    srcb = edges_buy[:, 0]
    dstb = edges_buy[:, 1]
    scores_c, scores_b = _sc_scores(embed_user, embed_item, srcc, dstc,
                                    srcb, dstb, w_click, w_buy)
    out = _tc_loss(scores_c.reshape(E // D, D), scores_b.reshape(E // D, D),
                   labels_click.reshape(E // D, D), labels_buy.reshape(E // D, D),
                   embed_user, embed_item,
                   w_click.reshape(1, D), w_buy.reshape(1, D))
    return out[0, 0]


# speedup vs baseline: 3.1949x; 3.1949x over previous
import functools

import jax
import jax.numpy as jnp
from jax import lax
from jax.experimental import pallas as pl
from jax.experimental.pallas import tpu as pltpu
from jax.experimental.pallas import tpu_sc as plsc

N = 10000
D = 128
E = 320000
REG_PARAM = 0.01

NC, NS = 2, 16
NW = NC * NS
EPW = E // NW
CH = 80
NCHUNK = EPW // CH


def _sc_scores_body(u_hbm, i_hbm, srcc_hbm, dstc_hbm, srcb_hbm, dstb_hbm,
                    wc_hbm, wb_hbm, outc_hbm, outb_hbm,
                    idxs, idxd, urows, irows, wbuf, scores, sem):
    wid = lax.axis_index("s") * NC + lax.axis_index("c")
    base = wid * EPW

    lane = lax.iota(jnp.int32, 16)

    def run_etype(src_hbm, dst_hbm, w_hbm, out_hbm):
        pltpu.sync_copy(w_hbm, wbuf)
        wv = [wbuf[pl.ds(16 * j, 16)] for j in range(8)]

        def chunk_body(c, carry):
            off = base + c * CH
            pltpu.sync_copy(src_hbm.at[pl.ds(off, CH)], idxs)
            pltpu.sync_copy(dst_hbm.at[pl.ds(off, CH)], idxd)
            cu = pltpu.async_copy(u_hbm.at[idxs], urows, sem)
            ci = pltpu.async_copy(i_hbm.at[idxd], irows, sem)
            cu.wait()
            ci.wait()

            def group_body(g, vec):
                for k in range(16):
                    e = g * 16 + k
                    acc = urows[e, pl.ds(0, 16)] * wv[0] * irows[e, pl.ds(0, 16)]
                    for j in range(1, 8):
                        acc = acc + (urows[e, pl.ds(16 * j, 16)] * wv[j]
                                     * irows[e, pl.ds(16 * j, 16)])
                    vec = jnp.where(lane == k, jnp.sum(acc), vec)
                scores[pl.ds(c * CH + g * 16, 16)] = vec
                return vec

            lax.fori_loop(0, CH // 16, group_body, jnp.zeros(16, jnp.float32))
            return carry

        lax.fori_loop(0, NCHUNK, chunk_body, 0)
        pltpu.sync_copy(scores, out_hbm.at[pl.ds(base, EPW)])

    run_etype(srcc_hbm, dstc_hbm, wc_hbm, outc_hbm)
    run_etype(srcb_hbm, dstb_hbm, wb_hbm, outb_hbm)


_sc_scores = pl.kernel(
    _sc_scores_body,
    out_type=(jax.ShapeDtypeStruct((E,), jnp.float32),
              jax.ShapeDtypeStruct((E,), jnp.float32)),
    mesh=plsc.VectorSubcoreMesh(core_axis_name="c", subcore_axis_name="s",
                                num_cores=NC, num_subcores=NS),
    scratch_types=[
        pltpu.VMEM((CH,), jnp.int32),
        pltpu.VMEM((CH,), jnp.int32),
        pltpu.VMEM((CH, D), jnp.float32),
        pltpu.VMEM((CH, D), jnp.float32),
        pltpu.VMEM((D,), jnp.float32),
        pltpu.VMEM((EPW,), jnp.float32),
        pltpu.SemaphoreType.DMA,
    ],
    compiler_params=pltpu.CompilerParams(needs_layout_passes=False),
)


def _tc_loss_body(sc_ref, sb_ref, lc_ref, lb_ref, u_ref, i_ref, wc_ref, wb_ref,
                  out_ref):
    def bce_sum(s, y):
        return jnp.sum(jnp.maximum(s, 0.0) - s * y
                       + jnp.log1p(jnp.exp(-jnp.abs(s))))

    predict = (bce_sum(sc_ref[...], lc_ref[...])
               + bce_sum(sb_ref[...], lb_ref[...])) / E
    reg = (jnp.mean(u_ref[...] ** 2) + jnp.mean(i_ref[...] ** 2)
           + jnp.mean(wc_ref[...] ** 2) + jnp.mean(wb_ref[...] ** 2))
    out_ref[...] = jnp.full((1, 1), predict + REG_PARAM * reg, jnp.float32)


_tc_loss = pl.pallas_call(
    _tc_loss_body,
    out_shape=jax.ShapeDtypeStruct((1, 1), jnp.float32),
)


def kernel(embed_user, embed_item, edges_click, edges_buy, labels_click,
           labels_buy, w_click, w_buy):
    srcc = edges_click[:, 0]
    dstc = edges_click[:, 1]
    srcb = edges_buy[:, 0]
    dstb = edges_buy[:, 1]
    scores_c, scores_b = _sc_scores(embed_user, embed_item, srcc, dstc,
                                    srcb, dstb, w_click, w_buy)
    out = _tc_loss(scores_c.reshape(E // D, D), scores_b.reshape(E // D, D),
                   labels_click.reshape(E // D, D), labels_buy.reshape(E // D, D),
                   embed_user, embed_item,
                   w_click.reshape(1, D), w_buy.reshape(1, D))
    return out[0, 0]


# speedup vs baseline: 4.4842x; 1.4036x over previous
import functools

import jax
import jax.numpy as jnp
from jax import lax
from jax.experimental import pallas as pl
from jax.experimental.pallas import tpu as pltpu
from jax.experimental.pallas import tpu_sc as plsc

N = 10000
D = 128
E = 320000
REG_PARAM = 0.01

NC, NS = 2, 16
NW = NC * NS
EPW = E // NW
CH = 80
NCHUNK = EPW // CH
NPAIR = (NCHUNK - 1) // 2


def _sc_scores_body(u_hbm, i_hbm, srcc_hbm, dstc_hbm, srcb_hbm, dstb_hbm,
                    wc_hbm, wb_hbm, outc_hbm, outb_hbm,
                    idxs, idxd, urowsA, irowsA, urowsB, irowsB,
                    wbuf, scores, semA, semB):
    wid = lax.axis_index("s") * NC + lax.axis_index("c")
    base = wid * EPW
    lane = lax.iota(jnp.int32, 16)

    def run_etype(src_hbm, dst_hbm, w_hbm, out_hbm):
        pltpu.sync_copy(w_hbm, wbuf)
        wv = [wbuf[pl.ds(16 * j, 16)] for j in range(8)]
        pltpu.sync_copy(src_hbm.at[pl.ds(base, EPW)], idxs)
        pltpu.sync_copy(dst_hbm.at[pl.ds(base, EPW)], idxd)

        def start(c, ubuf, ibuf, sem):
            pltpu.async_copy(u_hbm.at[idxs.at[pl.ds(c * CH, CH)]], ubuf, sem)
            pltpu.async_copy(i_hbm.at[idxd.at[pl.ds(c * CH, CH)]], ibuf, sem)

        def drain(ubuf, ibuf, sem):
            pltpu.make_async_copy(u_hbm.at[idxs.at[pl.ds(0, CH)]], ubuf, sem).wait()
            pltpu.make_async_copy(i_hbm.at[idxd.at[pl.ds(0, CH)]], ibuf, sem).wait()

        def compute(c, ubuf, ibuf):
            def group_body(g, vec):
                for k in range(16):
                    e = g * 16 + k
                    acc = ubuf[e, pl.ds(0, 16)] * wv[0] * ibuf[e, pl.ds(0, 16)]
                    for j in range(1, 8):
                        acc = acc + (ubuf[e, pl.ds(16 * j, 16)] * wv[j]
                                     * ibuf[e, pl.ds(16 * j, 16)])
                    vec = jnp.where(lane == k, jnp.sum(acc), vec)
                scores[pl.ds(c * CH + g * 16, 16)] = vec
                return vec

            lax.fori_loop(0, CH // 16, group_body, jnp.zeros(16, jnp.float32))

        start(0, urowsA, irowsA, semA)

        def pair_body(t, carry):
            c = 2 * t
            start(c + 1, urowsB, irowsB, semB)
            drain(urowsA, irowsA, semA)
            compute(c, urowsA, irowsA)
            start(c + 2, urowsA, irowsA, semA)
            drain(urowsB, irowsB, semB)
            compute(c + 1, urowsB, irowsB)
            return carry

        lax.fori_loop(0, NPAIR, pair_body, 0)
        drain(urowsA, irowsA, semA)
        compute(NCHUNK - 1, urowsA, irowsA)
        pltpu.sync_copy(scores, out_hbm.at[pl.ds(base, EPW)])

    run_etype(srcc_hbm, dstc_hbm, wc_hbm, outc_hbm)
    run_etype(srcb_hbm, dstb_hbm, wb_hbm, outb_hbm)


_sc_scores = pl.kernel(
    _sc_scores_body,
    out_type=(jax.ShapeDtypeStruct((E,), jnp.float32),
              jax.ShapeDtypeStruct((E,), jnp.float32)),
    mesh=plsc.VectorSubcoreMesh(core_axis_name="c", subcore_axis_name="s",
                                num_cores=NC, num_subcores=NS),
    scratch_types=[
        pltpu.VMEM((EPW,), jnp.int32),
        pltpu.VMEM((EPW,), jnp.int32),
        pltpu.VMEM((CH, D), jnp.float32),
        pltpu.VMEM((CH, D), jnp.float32),
        pltpu.VMEM((CH, D), jnp.float32),
        pltpu.VMEM((CH, D), jnp.float32),
        pltpu.VMEM((D,), jnp.float32),
        pltpu.VMEM((EPW,), jnp.float32),
        pltpu.SemaphoreType.DMA,
        pltpu.SemaphoreType.DMA,
    ],
    compiler_params=pltpu.CompilerParams(needs_layout_passes=False),
)


def _tc_loss_body(sc_ref, sb_ref, lc_ref, lb_ref, u_ref, i_ref, wc_ref, wb_ref,
                  out_ref):
    def bce_sum(s, y):
        return jnp.sum(jnp.maximum(s, 0.0) - s * y
                       + jnp.log1p(jnp.exp(-jnp.abs(s))))

    predict = (bce_sum(sc_ref[...], lc_ref[...])
               + bce_sum(sb_ref[...], lb_ref[...])) / E
    reg = (jnp.mean(u_ref[...] ** 2) + jnp.mean(i_ref[...] ** 2)
           + jnp.mean(wc_ref[...] ** 2) + jnp.mean(wb_ref[...] ** 2))
    out_ref[...] = jnp.full((1, 1), predict + REG_PARAM * reg, jnp.float32)


_tc_loss = pl.pallas_call(
    _tc_loss_body,
    out_shape=jax.ShapeDtypeStruct((1, 1), jnp.float32),
)


def kernel(embed_user, embed_item, edges_click, edges_buy, labels_click,
           labels_buy, w_click, w_buy):
    srcc = edges_click[:, 0]
    dstc = edges_click[:, 1]
    srcb = edges_buy[:, 0]
    dstb = edges_buy[:, 1]
    scores_c, scores_b = _sc_scores(embed_user, embed_item, srcc, dstc,
                                    srcb, dstb, w_click, w_buy)
    out = _tc_loss(scores_c.reshape(E // D, D), scores_b.reshape(E // D, D),
                   labels_click.reshape(E // D, D), labels_buy.reshape(E // D, D),
                   embed_user, embed_item,
                   w_click.reshape(1, D), w_buy.reshape(1, D))
    return out[0, 0]


# speedup vs baseline: 8.6597x; 1.9312x over previous
import functools

import jax
import jax.numpy as jnp
from jax import lax
from jax.experimental import pallas as pl
from jax.experimental.pallas import tpu as pltpu
from jax.experimental.pallas import tpu_sc as plsc

N = 10000
D = 128
E = 320000
REG_PARAM = 0.01

NC, NS = 2, 16
NW = NC * NS
EPW = E // NW
CH = 80
NCHUNK = EPW // CH
NPAIR = (NCHUNK - 1) // 2


def _sc_scores_body(uwc_hbm, uwb_hbm, i_hbm, srcc_hbm, dstc_hbm,
                    srcb_hbm, dstb_hbm, outc_hbm, outb_hbm,
                    idxs, idxd, urowsA, irowsA, urowsB, irowsB,
                    scores, semA, semB):
    wid = lax.axis_index("s") * NC + lax.axis_index("c")
    base = wid * EPW
    lane = lax.iota(jnp.int32, 16)
    zf = jnp.zeros(16, jnp.float32)

    def run_etype(uw_hbm, src_hbm, dst_hbm, out_hbm):
        pltpu.sync_copy(src_hbm.at[pl.ds(base, EPW)], idxs)
        pltpu.sync_copy(dst_hbm.at[pl.ds(base, EPW)], idxd)

        def start(c, ubuf, ibuf, sem):
            pltpu.async_copy(uw_hbm.at[idxs.at[pl.ds(c * CH, CH)]], ubuf, sem)
            pltpu.async_copy(i_hbm.at[idxd.at[pl.ds(c * CH, CH)]], ibuf, sem)

        def drain(ubuf, ibuf, sem):
            pltpu.make_async_copy(uw_hbm.at[idxs.at[pl.ds(0, CH)]], ubuf, sem).wait()
            pltpu.make_async_copy(i_hbm.at[idxd.at[pl.ds(0, CH)]], ibuf, sem).wait()

        def compute(c, ubuf, ibuf):
            def group_body(g, gcarry):
                row = g * 16 + lane

                def d_body(d, carry):
                    acc, col = carry
                    u16 = plsc.load_gather(ubuf, [row, col])
                    i16 = plsc.load_gather(ibuf, [row, col])
                    return (acc + u16 * i16, (col + 1) & (D - 1))

                acc, _ = lax.fori_loop(0, D, d_body, (zf, lane), unroll=8)
                scores[pl.ds(c * CH + g * 16, 16)] = acc
                return gcarry

            lax.fori_loop(0, CH // 16, group_body, 0)

        start(0, urowsA, irowsA, semA)

        def pair_body(t, carry):
            c = 2 * t
            start(c + 1, urowsB, irowsB, semB)
            drain(urowsA, irowsA, semA)
            compute(c, urowsA, irowsA)
            start(c + 2, urowsA, irowsA, semA)
            drain(urowsB, irowsB, semB)
            compute(c + 1, urowsB, irowsB)
            return carry

        lax.fori_loop(0, NPAIR, pair_body, 0)
        drain(urowsA, irowsA, semA)
        compute(NCHUNK - 1, urowsA, irowsA)
        pltpu.sync_copy(scores, out_hbm.at[pl.ds(base, EPW)])

    run_etype(uwc_hbm, srcc_hbm, dstc_hbm, outc_hbm)
    run_etype(uwb_hbm, srcb_hbm, dstb_hbm, outb_hbm)


_sc_scores = pl.kernel(
    _sc_scores_body,
    out_type=(jax.ShapeDtypeStruct((E,), jnp.float32),
              jax.ShapeDtypeStruct((E,), jnp.float32)),
    mesh=plsc.VectorSubcoreMesh(core_axis_name="c", subcore_axis_name="s",
                                num_cores=NC, num_subcores=NS),
    scratch_types=[
        pltpu.VMEM((EPW,), jnp.int32),
        pltpu.VMEM((EPW,), jnp.int32),
        pltpu.VMEM((CH, D), jnp.float32),
        pltpu.VMEM((CH, D), jnp.float32),
        pltpu.VMEM((CH, D), jnp.float32),
        pltpu.VMEM((CH, D), jnp.float32),
        pltpu.VMEM((EPW,), jnp.float32),
        pltpu.SemaphoreType.DMA,
        pltpu.SemaphoreType.DMA,
    ],
    compiler_params=pltpu.CompilerParams(needs_layout_passes=False),
)


def _tc_prescale_body(u_ref, wc_ref, wb_ref, uwc_ref, uwb_ref):
    u = u_ref[...]
    uwc_ref[...] = u * wc_ref[...]
    uwb_ref[...] = u * wb_ref[...]


_tc_prescale = pl.pallas_call(
    _tc_prescale_body,
    out_shape=(jax.ShapeDtypeStruct((N, D), jnp.float32),
               jax.ShapeDtypeStruct((N, D), jnp.float32)),
)


def _tc_loss_body(sc_ref, sb_ref, lc_ref, lb_ref, u_ref, i_ref, wc_ref, wb_ref,
                  out_ref):
    def bce_sum(s, y):
        return jnp.sum(jnp.maximum(s, 0.0) - s * y
                       + jnp.log1p(jnp.exp(-jnp.abs(s))))

    predict = (bce_sum(sc_ref[...], lc_ref[...])
               + bce_sum(sb_ref[...], lb_ref[...])) / E
    reg = (jnp.mean(u_ref[...] ** 2) + jnp.mean(i_ref[...] ** 2)
           + jnp.mean(wc_ref[...] ** 2) + jnp.mean(wb_ref[...] ** 2))
    out_ref[...] = jnp.full((1, 1), predict + REG_PARAM * reg, jnp.float32)


_tc_loss = pl.pallas_call(
    _tc_loss_body,
    out_shape=jax.ShapeDtypeStruct((1, 1), jnp.float32),
)


def kernel(embed_user, embed_item, edges_click, edges_buy, labels_click,
           labels_buy, w_click, w_buy):
    srcc = edges_click[:, 0]
    dstc = edges_click[:, 1]
    srcb = edges_buy[:, 0]
    dstb = edges_buy[:, 1]
    uwc, uwb = _tc_prescale(embed_user, w_click.reshape(1, D),
                            w_buy.reshape(1, D))
    scores_c, scores_b = _sc_scores(uwc, uwb, embed_item, srcc, dstc,
                                    srcb, dstb)
    out = _tc_loss(scores_c.reshape(E // D, D), scores_b.reshape(E // D, D),
                   labels_click.reshape(E // D, D), labels_buy.reshape(E // D, D),
                   embed_user, embed_item,
                   w_click.reshape(1, D), w_buy.reshape(1, D))
    return out[0, 0]


# speedup vs baseline: 8.7925x; 1.0153x over previous
import functools

import jax
import jax.numpy as jnp
from jax import lax
from jax.experimental import pallas as pl
from jax.experimental.pallas import tpu as pltpu
from jax.experimental.pallas import tpu_sc as plsc

N = 10000
D = 128
DP = D // 2
E = 320000
REG_PARAM = 0.01

NC, NS = 2, 16
NW = NC * NS
EPW = E // NW
CH = 80
NCHUNK = EPW // CH
NPAIR = (NCHUNK - 1) // 2


def _sc_scores_body(uwc_hbm, uwb_hbm, i_hbm, srcc_hbm, dstc_hbm,
                    srcb_hbm, dstb_hbm, outc_hbm, outb_hbm,
                    idxs, idxd, urowsA, irowsA, urowsB, irowsB,
                    scores, semA, semB):
    wid = lax.axis_index("s") * NC + lax.axis_index("c")
    base = wid * EPW
    lane = lax.iota(jnp.int32, 16)
    zbf = jnp.zeros(32, jnp.bfloat16)

    def run_etype(uw_hbm, src_hbm, dst_hbm, out_hbm):
        pltpu.sync_copy(src_hbm.at[pl.ds(base, EPW)], idxs)
        pltpu.sync_copy(dst_hbm.at[pl.ds(base, EPW)], idxd)

        def start(c, ubuf, ibuf, sem):
            pltpu.async_copy(uw_hbm.at[idxs.at[pl.ds(c * CH, CH)]], ubuf, sem)
            pltpu.async_copy(i_hbm.at[idxd.at[pl.ds(c * CH, CH)]], ibuf, sem)

        def drain(ubuf, ibuf, sem):
            pltpu.make_async_copy(uw_hbm.at[idxs.at[pl.ds(0, CH)]], ubuf, sem).wait()
            pltpu.make_async_copy(i_hbm.at[idxd.at[pl.ds(0, CH)]], ibuf, sem).wait()

        def compute(c, ubuf, ibuf):
            def group_body(g, gcarry):
                row = g * 16 + lane

                def d_body(d, carry):
                    acc, col = carry
                    up = plsc.load_gather(ubuf, [row, col])
                    ip = plsc.load_gather(ibuf, [row, col])
                    acc = acc + (plsc.bitcast(up, jnp.bfloat16)
                                 * plsc.bitcast(ip, jnp.bfloat16))
                    return (acc, (col + 1) & (DP - 1))

                acc, _ = lax.fori_loop(0, DP, d_body, (zbf, lane), unroll=8)
                hi, lo = plsc.unpack(acc, format=plsc.PackFormat.INTERLEAVED)
                scores[pl.ds(c * CH + g * 16, 16)] = hi + lo
                return gcarry

            lax.fori_loop(0, CH // 16, group_body, 0)

        start(0, urowsA, irowsA, semA)

        def pair_body(t, carry):
            c = 2 * t
            start(c + 1, urowsB, irowsB, semB)
            drain(urowsA, irowsA, semA)
            compute(c, urowsA, irowsA)
            start(c + 2, urowsA, irowsA, semA)
            drain(urowsB, irowsB, semB)
            compute(c + 1, urowsB, irowsB)
            return carry

        lax.fori_loop(0, NPAIR, pair_body, 0)
        drain(urowsA, irowsA, semA)
        compute(NCHUNK - 1, urowsA, irowsA)
        pltpu.sync_copy(scores, out_hbm.at[pl.ds(base, EPW)])

    run_etype(uwc_hbm, srcc_hbm, dstc_hbm, outc_hbm)
    run_etype(uwb_hbm, srcb_hbm, dstb_hbm, outb_hbm)


_sc_scores = pl.kernel(
    _sc_scores_body,
    out_type=(jax.ShapeDtypeStruct((E,), jnp.float32),
              jax.ShapeDtypeStruct((E,), jnp.float32)),
    mesh=plsc.VectorSubcoreMesh(core_axis_name="c", subcore_axis_name="s",
                                num_cores=NC, num_subcores=NS),
    scratch_types=[
        pltpu.VMEM((EPW,), jnp.int32),
        pltpu.VMEM((EPW,), jnp.int32),
        pltpu.VMEM((CH, DP), jnp.int32),
        pltpu.VMEM((CH, DP), jnp.int32),
        pltpu.VMEM((CH, DP), jnp.int32),
        pltpu.VMEM((CH, DP), jnp.int32),
        pltpu.VMEM((EPW,), jnp.float32),
        pltpu.SemaphoreType.DMA,
        pltpu.SemaphoreType.DMA,
    ],
    compiler_params=pltpu.CompilerParams(needs_layout_passes=False,
                                         use_tc_tiling_on_sc=False),
)


def _tc_prescale_body(u_ref, i_ref, wc_ref, wb_ref, uwc_ref, uwb_ref, ib_ref):
    u = u_ref[...]
    uwc_ref[...] = (u * wc_ref[...]).astype(jnp.bfloat16)
    uwb_ref[...] = (u * wb_ref[...]).astype(jnp.bfloat16)
    ib_ref[...] = i_ref[...].astype(jnp.bfloat16)


_tc_prescale = pl.pallas_call(
    _tc_prescale_body,
    out_shape=(jax.ShapeDtypeStruct((N, D), jnp.bfloat16),
               jax.ShapeDtypeStruct((N, D), jnp.bfloat16),
               jax.ShapeDtypeStruct((N, D), jnp.bfloat16)),
)


def _tc_loss_body(sc_ref, sb_ref, lc_ref, lb_ref, u_ref, i_ref, wc_ref, wb_ref,
                  out_ref):
    def bce_sum(s, y):
        return jnp.sum(jnp.maximum(s, 0.0) - s * y
                       + jnp.log1p(jnp.exp(-jnp.abs(s))))

    predict = (bce_sum(sc_ref[...], lc_ref[...])
               + bce_sum(sb_ref[...], lb_ref[...])) / E
    reg = (jnp.mean(u_ref[...] ** 2) + jnp.mean(i_ref[...] ** 2)
           + jnp.mean(wc_ref[...] ** 2) + jnp.mean(wb_ref[...] ** 2))
    out_ref[...] = jnp.full((1, 1), predict + REG_PARAM * reg, jnp.float32)


_tc_loss = pl.pallas_call(
    _tc_loss_body,
    out_shape=jax.ShapeDtypeStruct((1, 1), jnp.float32),
)


def _pack(x_bf16):
    return jax.lax.bitcast_convert_type(x_bf16.reshape(N, DP, 2), jnp.int32)


def kernel(embed_user, embed_item, edges_click, edges_buy, labels_click,
           labels_buy, w_click, w_buy):
    srcc = edges_click[:, 0]
    dstc = edges_click[:, 1]
    srcb = edges_buy[:, 0]
    dstb = edges_buy[:, 1]
    uwc, uwb, ib = _tc_prescale(embed_user, embed_item,
                                w_click.reshape(1, D), w_buy.reshape(1, D))
    scores_c, scores_b = _sc_scores(_pack(uwc), _pack(uwb), _pack(ib),
                                    srcc, dstc, srcb, dstb)
    out = _tc_loss(scores_c.reshape(E // D, D), scores_b.reshape(E // D, D),
                   labels_click.reshape(E // D, D), labels_buy.reshape(E // D, D),
                   embed_user, embed_item,
                   w_click.reshape(1, D), w_buy.reshape(1, D))
    return out[0, 0]


# speedup vs baseline: 9.4824x; 1.0785x over previous
import functools

import jax
import jax.numpy as jnp
from jax import lax
from jax.experimental import pallas as pl
from jax.experimental.pallas import tpu as pltpu
from jax.experimental.pallas import tpu_sc as plsc

N = 10000
D = 128
DP = D // 2
E = 320000
REG_PARAM = 0.01

NC, NS = 2, 16
NW = NC * NS
EPW = E // NW
CH = 80
NCHUNK = EPW // CH
NPAIR = (NCHUNK - 1) // 2


def _sc_scores_body(uwc_hbm, uwb_hbm, i_hbm, srcc_hbm, dstc_hbm,
                    srcb_hbm, dstb_hbm, outc_hbm, outb_hbm,
                    idxs, idxd, urowsA, irowsA, urowsB, irowsB,
                    scores, semA, semB):
    wid = lax.axis_index("s") * NC + lax.axis_index("c")
    base = wid * EPW
    lane = lax.iota(jnp.int32, 16)
    zbf = jnp.zeros(32, jnp.bfloat16)

    def run_etype(uw_hbm, src_hbm, dst_hbm, out_hbm):
        pltpu.sync_copy(src_hbm.at[pl.ds(base, EPW)], idxs)
        pltpu.sync_copy(dst_hbm.at[pl.ds(base, EPW)], idxd)

        def start(c, ubuf, ibuf, sem):
            pltpu.async_copy(uw_hbm.at[idxs.at[pl.ds(c * CH, CH)]], ubuf, sem)
            pltpu.async_copy(i_hbm.at[idxd.at[pl.ds(c * CH, CH)]], ibuf, sem)

        def drain(ubuf, ibuf, sem):
            pltpu.make_async_copy(uw_hbm.at[idxs.at[pl.ds(0, CH)]], ubuf, sem).wait()
            pltpu.make_async_copy(i_hbm.at[idxd.at[pl.ds(0, CH)]], ibuf, sem).wait()

        def compute(c, ubuf, ibuf):
            def group_body(g, gcarry):
                row = g * 16 + lane

                def d_body(d, carry):
                    acc, col = carry
                    up = plsc.load_gather(ubuf, [row, col])
                    ip = plsc.load_gather(ibuf, [row, col])
                    acc = acc + (plsc.bitcast(up, jnp.bfloat16)
                                 * plsc.bitcast(ip, jnp.bfloat16))
                    return (acc, (col + 1) & (DP - 1))

                acc, _ = lax.fori_loop(0, DP, d_body, (zbf, lane), unroll=8)
                hi, lo = plsc.unpack(acc, format=plsc.PackFormat.INTERLEAVED)
                scores[pl.ds(c * CH + g * 16, 16)] = hi + lo
                return gcarry

            lax.fori_loop(0, CH // 16, group_body, 0)

        start(0, urowsA, irowsA, semA)

        def pair_body(t, carry):
            c = 2 * t
            start(c + 1, urowsB, irowsB, semB)
            drain(urowsA, irowsA, semA)
            start(c + 2, urowsA, irowsA, semA)
            drain(urowsB, irowsB, semB)
            return carry

        lax.fori_loop(0, NPAIR, pair_body, 0)
        drain(urowsA, irowsA, semA)
        compute(NCHUNK - 1, urowsA, irowsA)
        pltpu.sync_copy(scores, out_hbm.at[pl.ds(base, EPW)])

    run_etype(uwc_hbm, srcc_hbm, dstc_hbm, outc_hbm)
    run_etype(uwb_hbm, srcb_hbm, dstb_hbm, outb_hbm)


_sc_scores = pl.kernel(
    _sc_scores_body,
    out_type=(jax.ShapeDtypeStruct((E,), jnp.float32),
              jax.ShapeDtypeStruct((E,), jnp.float32)),
    mesh=plsc.VectorSubcoreMesh(core_axis_name="c", subcore_axis_name="s",
                                num_cores=NC, num_subcores=NS),
    scratch_types=[
        pltpu.VMEM((EPW,), jnp.int32),
        pltpu.VMEM((EPW,), jnp.int32),
        pltpu.VMEM((CH, DP), jnp.int32),
        pltpu.VMEM((CH, DP), jnp.int32),
        pltpu.VMEM((CH, DP), jnp.int32),
        pltpu.VMEM((CH, DP), jnp.int32),
        pltpu.VMEM((EPW,), jnp.float32),
        pltpu.SemaphoreType.DMA,
        pltpu.SemaphoreType.DMA,
    ],
    compiler_params=pltpu.CompilerParams(needs_layout_passes=False,
                                         use_tc_tiling_on_sc=False),
)


def _tc_prescale_body(u_ref, i_ref, wc_ref, wb_ref, uwc_ref, uwb_ref, ib_ref):
    u = u_ref[...]
    uwc_ref[...] = (u * wc_ref[...]).astype(jnp.bfloat16)
    uwb_ref[...] = (u * wb_ref[...]).astype(jnp.bfloat16)
    ib_ref[...] = i_ref[...].astype(jnp.bfloat16)


_tc_prescale = pl.pallas_call(
    _tc_prescale_body,
    out_shape=(jax.ShapeDtypeStruct((N, D), jnp.bfloat16),
               jax.ShapeDtypeStruct((N, D), jnp.bfloat16),
               jax.ShapeDtypeStruct((N, D), jnp.bfloat16)),
)


def _tc_loss_body(sc_ref, sb_ref, lc_ref, lb_ref, u_ref, i_ref, wc_ref, wb_ref,
                  out_ref):
    def bce_sum(s, y):
        return jnp.sum(jnp.maximum(s, 0.0) - s * y
                       + jnp.log1p(jnp.exp(-jnp.abs(s))))

    predict = (bce_sum(sc_ref[...], lc_ref[...])
               + bce_sum(sb_ref[...], lb_ref[...])) / E
    reg = (jnp.mean(u_ref[...] ** 2) + jnp.mean(i_ref[...] ** 2)
           + jnp.mean(wc_ref[...] ** 2) + jnp.mean(wb_ref[...] ** 2))
    out_ref[...] = jnp.full((1, 1), predict + REG_PARAM * reg, jnp.float32)


_tc_loss = pl.pallas_call(
    _tc_loss_body,
    out_shape=jax.ShapeDtypeStruct((1, 1), jnp.float32),
)


def _pack(x_bf16):
    return jax.lax.bitcast_convert_type(x_bf16.reshape(N, DP, 2), jnp.int32)


def kernel(embed_user, embed_item, edges_click, edges_buy, labels_click,
           labels_buy, w_click, w_buy):
    srcc = edges_click[:, 0]
    dstc = edges_click[:, 1]
    srcb = edges_buy[:, 0]
    dstb = edges_buy[:, 1]
    uwc, uwb, ib = _tc_prescale(embed_user, embed_item,
                                w_click.reshape(1, D), w_buy.reshape(1, D))
    scores_c, scores_b = _sc_scores(_pack(uwc), _pack(uwb), _pack(ib),
                                    srcc, dstc, srcb, dstb)
    out = _tc_loss(scores_c.reshape(E // D, D), scores_b.reshape(E // D, D),
                   labels_click.reshape(E // D, D), labels_buy.reshape(E // D, D),
                   embed_user, embed_item,
                   w_click.reshape(1, D), w_buy.reshape(1, D))
    return out[0, 0]


# speedup vs baseline: 10.3948x; 1.0962x over previous
import functools

import jax
import jax.numpy as jnp
from jax import lax
from jax.experimental import pallas as pl
from jax.experimental.pallas import tpu as pltpu
from jax.experimental.pallas import tpu_sc as plsc

N = 10000
D = 128
DP = D // 2
E = 320000
REG_PARAM = 0.01

NC, NS = 2, 16
NW = NC * NS
EPW = E // NW
CH = 80
NCHUNK = EPW // CH
NPAIR = (NCHUNK - 1) // 2


def _sc_scores_body(uwc_hbm, uwb_hbm, i_hbm, srcc_hbm, dstc_hbm,
                    srcb_hbm, dstb_hbm, outc_hbm, outb_hbm,
                    idxs, idxd, urowsA, irowsA, urowsB, irowsB,
                    urowsC, irowsC, urowsD, irowsD,
                    scores, semA, semB, semC, semD):
    wid = lax.axis_index("s") * NC + lax.axis_index("c")
    base = wid * EPW
    lane = lax.iota(jnp.int32, 16)
    zbf = jnp.zeros(32, jnp.bfloat16)

    def run_etype(uw_hbm, src_hbm, dst_hbm, out_hbm):
        pltpu.sync_copy(src_hbm.at[pl.ds(base, EPW)], idxs)
        pltpu.sync_copy(dst_hbm.at[pl.ds(base, EPW)], idxd)

        def start(c, ubuf, ibuf, sem):
            pltpu.async_copy(uw_hbm.at[idxs.at[pl.ds(c * CH, CH)]], ubuf, sem)
            pltpu.async_copy(i_hbm.at[idxd.at[pl.ds(c * CH, CH)]], ibuf, sem)

        def drain(ubuf, ibuf, sem):
            pltpu.make_async_copy(uw_hbm.at[idxs.at[pl.ds(0, CH)]], ubuf, sem).wait()
            pltpu.make_async_copy(i_hbm.at[idxd.at[pl.ds(0, CH)]], ibuf, sem).wait()

        def compute(c, ubuf, ibuf):
            def group_body(g, gcarry):
                row = g * 16 + lane

                def d_body(d, carry):
                    acc, col = carry
                    up = plsc.load_gather(ubuf, [row, col])
                    ip = plsc.load_gather(ibuf, [row, col])
                    acc = acc + (plsc.bitcast(up, jnp.bfloat16)
                                 * plsc.bitcast(ip, jnp.bfloat16))
                    return (acc, (col + 1) & (DP - 1))

                acc, _ = lax.fori_loop(0, DP, d_body, (zbf, lane), unroll=8)
                hi, lo = plsc.unpack(acc, format=plsc.PackFormat.INTERLEAVED)
                scores[pl.ds(c * CH + g * 16, 16)] = hi + lo
                return gcarry

            lax.fori_loop(0, CH // 16, group_body, 0)

        start(0, urowsA, irowsA, semA)
        start(1, urowsB, irowsB, semB)
        start(2, urowsC, irowsC, semC)

        def quad_body(t, carry):
            c = 4 * t
            start(c + 3, urowsD, irowsD, semD)
            drain(urowsA, irowsA, semA)
            start(c + 4, urowsA, irowsA, semA)
            drain(urowsB, irowsB, semB)
            start(c + 5, urowsB, irowsB, semB)
            drain(urowsC, irowsC, semC)
            start(c + 6, urowsC, irowsC, semC)
            drain(urowsD, irowsD, semD)
            return carry

        lax.fori_loop(0, 30, quad_body, 0)
        start(123, urowsD, irowsD, semD)
        start(124, urowsB, irowsB, semB)
        drain(urowsA, irowsA, semA)
        drain(urowsD, irowsD, semD)
        drain(urowsB, irowsB, semB)
        compute(NCHUNK - 1, urowsB, irowsB)
        pltpu.sync_copy(scores, out_hbm.at[pl.ds(base, EPW)])

    run_etype(uwc_hbm, srcc_hbm, dstc_hbm, outc_hbm)
    run_etype(uwb_hbm, srcb_hbm, dstb_hbm, outb_hbm)


_sc_scores = pl.kernel(
    _sc_scores_body,
    out_type=(jax.ShapeDtypeStruct((E,), jnp.float32),
              jax.ShapeDtypeStruct((E,), jnp.float32)),
    mesh=plsc.VectorSubcoreMesh(core_axis_name="c", subcore_axis_name="s",
                                num_cores=NC, num_subcores=NS),
    scratch_types=[
        pltpu.VMEM((EPW,), jnp.int32),
        pltpu.VMEM((EPW,), jnp.int32),
        pltpu.VMEM((CH, DP), jnp.int32),
        pltpu.VMEM((CH, DP), jnp.int32),
        pltpu.VMEM((CH, DP), jnp.int32),
        pltpu.VMEM((CH, DP), jnp.int32),
        pltpu.VMEM((CH, DP), jnp.int32),
        pltpu.VMEM((CH, DP), jnp.int32),
        pltpu.VMEM((CH, DP), jnp.int32),
        pltpu.VMEM((CH, DP), jnp.int32),
        pltpu.VMEM((EPW,), jnp.float32),
        pltpu.SemaphoreType.DMA,
        pltpu.SemaphoreType.DMA,
        pltpu.SemaphoreType.DMA,
        pltpu.SemaphoreType.DMA,
    ],
    compiler_params=pltpu.CompilerParams(needs_layout_passes=False,
                                         use_tc_tiling_on_sc=False),
)


def _tc_prescale_body(u_ref, i_ref, wc_ref, wb_ref, uwc_ref, uwb_ref, ib_ref):
    u = u_ref[...]
    uwc_ref[...] = (u * wc_ref[...]).astype(jnp.bfloat16)
    uwb_ref[...] = (u * wb_ref[...]).astype(jnp.bfloat16)
    ib_ref[...] = i_ref[...].astype(jnp.bfloat16)


_tc_prescale = pl.pallas_call(
    _tc_prescale_body,
    out_shape=(jax.ShapeDtypeStruct((N, D), jnp.bfloat16),
               jax.ShapeDtypeStruct((N, D), jnp.bfloat16),
               jax.ShapeDtypeStruct((N, D), jnp.bfloat16)),
)


def _tc_loss_body(sc_ref, sb_ref, lc_ref, lb_ref, u_ref, i_ref, wc_ref, wb_ref,
                  out_ref):
    def bce_sum(s, y):
        return jnp.sum(jnp.maximum(s, 0.0) - s * y
                       + jnp.log1p(jnp.exp(-jnp.abs(s))))

    predict = (bce_sum(sc_ref[...], lc_ref[...])
               + bce_sum(sb_ref[...], lb_ref[...])) / E
    reg = (jnp.mean(u_ref[...] ** 2) + jnp.mean(i_ref[...] ** 2)
           + jnp.mean(wc_ref[...] ** 2) + jnp.mean(wb_ref[...] ** 2))
    out_ref[...] = jnp.full((1, 1), predict + REG_PARAM * reg, jnp.float32)


_tc_loss = pl.pallas_call(
    _tc_loss_body,
    out_shape=jax.ShapeDtypeStruct((1, 1), jnp.float32),
)


def _pack(x_bf16):
    return jax.lax.bitcast_convert_type(x_bf16.reshape(N, DP, 2), jnp.int32)


def kernel(embed_user, embed_item, edges_click, edges_buy, labels_click,
           labels_buy, w_click, w_buy):
    srcc = edges_click[:, 0]
    dstc = edges_click[:, 1]
    srcb = edges_buy[:, 0]
    dstb = edges_buy[:, 1]
    uwc, uwb, ib = _tc_prescale(embed_user, embed_item,
                                w_click.reshape(1, D), w_buy.reshape(1, D))
    scores_c, scores_b = _sc_scores(_pack(uwc), _pack(uwb), _pack(ib),
                                    srcc, dstc, srcb, dstb)
    out = _tc_loss(scores_c.reshape(E // D, D), scores_b.reshape(E // D, D),
                   labels_click.reshape(E // D, D), labels_buy.reshape(E // D, D),
                   embed_user, embed_item,
                   w_click.reshape(1, D), w_buy.reshape(1, D))
    return out[0, 0]


# speedup vs baseline: 10.8880x; 1.0474x over previous
import functools

import jax
import jax.numpy as jnp
from jax import lax
from jax.experimental import pallas as pl
from jax.experimental.pallas import tpu as pltpu
from jax.experimental.pallas import tpu_sc as plsc

N = 10000
D = 128
DP = D // 2
E = 320000
REG_PARAM = 0.01

NC, NS = 2, 16
NW = NC * NS
EPW = E // NW
CH = 80
NCHUNK = EPW // CH
NPAIR = (NCHUNK - 1) // 2


def _sc_scores_body(uwc_hbm, uwb_hbm, i_hbm, srcc_hbm, dstc_hbm,
                    srcb_hbm, dstb_hbm, outc_hbm, outb_hbm,
                    idxs, idxd, urowsA, irowsA, urowsB, irowsB,
                    urowsC, irowsC, urowsD, irowsD,
                    scores, semA, semB, semC, semD):
    wid = lax.axis_index("s") * NC + lax.axis_index("c")
    base = wid * EPW
    lane = lax.iota(jnp.int32, 16)
    zbf = jnp.zeros(32, jnp.bfloat16)

    def run_etype(uw_hbm, src_hbm, dst_hbm, out_hbm):
        pltpu.sync_copy(src_hbm.at[pl.ds(base, EPW)], idxs)
        pltpu.sync_copy(dst_hbm.at[pl.ds(base, EPW)], idxd)

        def start(c, ubuf, ibuf, sem):
            pltpu.async_copy(uw_hbm.at[idxs.at[pl.ds(c * CH, CH)]], ubuf, sem)
            pltpu.async_copy(i_hbm.at[idxd.at[pl.ds(c * CH, CH)]], ibuf, sem)

        def drain(ubuf, ibuf, sem):
            pltpu.make_async_copy(uw_hbm.at[idxs.at[pl.ds(0, CH)]], ubuf, sem).wait()
            pltpu.make_async_copy(i_hbm.at[idxd.at[pl.ds(0, CH)]], ibuf, sem).wait()

        def compute(c, ubuf, ibuf):
            def group_body(g, gcarry):
                row = g * 16 + lane

                def d_body(d, carry):
                    acc, col = carry
                    up = plsc.load_gather(ubuf, [row, col])
                    ip = plsc.load_gather(ibuf, [row, col])
                    acc = acc + (plsc.bitcast(up, jnp.bfloat16)
                                 * plsc.bitcast(ip, jnp.bfloat16))
                    return (acc, (col + 1) & (DP - 1))

                acc, _ = lax.fori_loop(0, DP, d_body, (zbf, lane), unroll=8)
                hi, lo = plsc.unpack(acc, format=plsc.PackFormat.INTERLEAVED)
                scores[pl.ds(c * CH + g * 16, 16)] = hi + lo
                return gcarry

            lax.fori_loop(0, CH // 16, group_body, 0)

        start(0, urowsA, irowsA, semA)
        start(1, urowsB, irowsB, semB)
        start(2, urowsC, irowsC, semC)

        def quad_body(t, carry):
            c = 4 * t
            start(c + 3, urowsD, irowsD, semD)
            drain(urowsA, irowsA, semA)
            compute(c, urowsA, irowsA)
            start(c + 4, urowsA, irowsA, semA)
            drain(urowsB, irowsB, semB)
            compute(c + 1, urowsB, irowsB)
            start(c + 5, urowsB, irowsB, semB)
            drain(urowsC, irowsC, semC)
            compute(c + 2, urowsC, irowsC)
            start(c + 6, urowsC, irowsC, semC)
            drain(urowsD, irowsD, semD)
            compute(c + 3, urowsD, irowsD)
            return carry

        lax.fori_loop(0, 30, quad_body, 0)
        start(123, urowsD, irowsD, semD)
        drain(urowsA, irowsA, semA)
        compute(120, urowsA, irowsA)
        start(124, urowsA, irowsA, semA)
        drain(urowsB, irowsB, semB)
        compute(121, urowsB, irowsB)
        drain(urowsC, irowsC, semC)
        compute(122, urowsC, irowsC)
        drain(urowsD, irowsD, semD)
        compute(123, urowsD, irowsD)
        drain(urowsA, irowsA, semA)
        compute(NCHUNK - 1, urowsA, irowsA)
        pltpu.sync_copy(scores, out_hbm.at[pl.ds(base, EPW)])

    run_etype(uwc_hbm, srcc_hbm, dstc_hbm, outc_hbm)
    run_etype(uwb_hbm, srcb_hbm, dstb_hbm, outb_hbm)


_sc_scores = pl.kernel(
    _sc_scores_body,
    out_type=(jax.ShapeDtypeStruct((E,), jnp.float32),
              jax.ShapeDtypeStruct((E,), jnp.float32)),
    mesh=plsc.VectorSubcoreMesh(core_axis_name="c", subcore_axis_name="s",
                                num_cores=NC, num_subcores=NS),
    scratch_types=[
        pltpu.VMEM((EPW,), jnp.int32),
        pltpu.VMEM((EPW,), jnp.int32),
        pltpu.VMEM((CH, DP), jnp.int32),
        pltpu.VMEM((CH, DP), jnp.int32),
        pltpu.VMEM((CH, DP), jnp.int32),
        pltpu.VMEM((CH, DP), jnp.int32),
        pltpu.VMEM((CH, DP), jnp.int32),
        pltpu.VMEM((CH, DP), jnp.int32),
        pltpu.VMEM((CH, DP), jnp.int32),
        pltpu.VMEM((CH, DP), jnp.int32),
        pltpu.VMEM((EPW,), jnp.float32),
        pltpu.SemaphoreType.DMA,
        pltpu.SemaphoreType.DMA,
        pltpu.SemaphoreType.DMA,
        pltpu.SemaphoreType.DMA,
    ],
    compiler_params=pltpu.CompilerParams(needs_layout_passes=False,
                                         use_tc_tiling_on_sc=False),
)


def _tc_prescale_body(u_ref, i_ref, wc_ref, wb_ref, uwc_ref, uwb_ref, ib_ref):
    u = u_ref[...]
    uwc_ref[...] = (u * wc_ref[...]).astype(jnp.bfloat16)
    uwb_ref[...] = (u * wb_ref[...]).astype(jnp.bfloat16)
    ib_ref[...] = i_ref[...].astype(jnp.bfloat16)


_tc_prescale = pl.pallas_call(
    _tc_prescale_body,
    out_shape=(jax.ShapeDtypeStruct((N, D), jnp.bfloat16),
               jax.ShapeDtypeStruct((N, D), jnp.bfloat16),
               jax.ShapeDtypeStruct((N, D), jnp.bfloat16)),
)


def _tc_loss_body(sc_ref, sb_ref, lc_ref, lb_ref, u_ref, i_ref, wc_ref, wb_ref,
                  out_ref):
    def bce_sum(s, y):
        return jnp.sum(jnp.maximum(s, 0.0) - s * y
                       + jnp.log1p(jnp.exp(-jnp.abs(s))))

    predict = (bce_sum(sc_ref[...], lc_ref[...])
               + bce_sum(sb_ref[...], lb_ref[...])) / E
    reg = (jnp.mean(u_ref[...] ** 2) + jnp.mean(i_ref[...] ** 2)
           + jnp.mean(wc_ref[...] ** 2) + jnp.mean(wb_ref[...] ** 2))
    out_ref[...] = jnp.full((1, 1), predict + REG_PARAM * reg, jnp.float32)


_tc_loss = pl.pallas_call(
    _tc_loss_body,
    out_shape=jax.ShapeDtypeStruct((1, 1), jnp.float32),
)


def _pack(x_bf16):
    return jax.lax.bitcast_convert_type(x_bf16.reshape(N, DP, 2), jnp.int32)


def kernel(embed_user, embed_item, edges_click, edges_buy, labels_click,
           labels_buy, w_click, w_buy):
    srcc = edges_click[:, 0]
    dstc = edges_click[:, 1]
    srcb = edges_buy[:, 0]
    dstb = edges_buy[:, 1]
    uwc, uwb, ib = _tc_prescale(embed_user, embed_item,
                                w_click.reshape(1, D), w_buy.reshape(1, D))
    scores_c, scores_b = _sc_scores(_pack(uwc), _pack(uwb), _pack(ib),
                                    srcc, dstc, srcb, dstb)
    out = _tc_loss(scores_c.reshape(E // D, D), scores_b.reshape(E // D, D),
                   labels_click.reshape(E // D, D), labels_buy.reshape(E // D, D),
                   embed_user, embed_item,
                   w_click.reshape(1, D), w_buy.reshape(1, D))
    return out[0, 0]
